# baseline (speedup 1.0000x reference)
import functools

import jax
import jax.numpy as jnp
from jax import lax
from jax.experimental import pallas as pl
from jax.experimental.pallas import tpu as pltpu
from jax.experimental.pallas import tpu_sc as plsc

NC = 2
NS = 16
NW = NC * NS
K = 64
L = 16


def _matmul(x, W):
    n, d_in = x.shape
    d_out = W.shape[1]
    blk = 1000

    def mm(x_ref, w_ref, o_ref):
        o_ref[...] = jnp.dot(x_ref[...], w_ref[...],
                             preferred_element_type=jnp.float32)

    return pl.pallas_call(
        mm,
        grid=(n // blk,),
        in_specs=[pl.BlockSpec((blk, d_in), lambda i: (i, 0)),
                  pl.BlockSpec((d_in, d_out), lambda i: (0, 0))],
        out_specs=pl.BlockSpec((blk, d_out), lambda i: (i, 0)),
        out_shape=jax.ShapeDtypeStruct((n, d_out), jnp.float32),
    )(x, W)


def _spmm_partials(rows, cols, vals, dense, n_pad, d):
    e_pad = rows.shape[0]
    epw = e_pad // NW
    n_chunks = epw // K
    assert n_chunks % 2 == 0 and n_chunks >= 4
    npt = n_pad // NS
    dv = d // L

    mesh = plsc.VectorSubcoreMesh(core_axis_name="c", subcore_axis_name="s")

    @functools.partial(
        pl.kernel,
        mesh=mesh,
        out_type=jax.ShapeDtypeStruct((NC, n_pad, d), jnp.float32),
        scratch_types=[
            pltpu.VMEM((epw,), jnp.int32),
            pltpu.VMEM((K,), jnp.int32),
            pltpu.VMEM((K,), jnp.int32),
            pltpu.VMEM((K,), jnp.float32),
            pltpu.VMEM((K,), jnp.float32),
            pltpu.VMEM((K, d), jnp.float32),
            pltpu.VMEM((K, d), jnp.float32),
            pltpu.VMEM_SHARED((n_pad, d), jnp.float32),
            pltpu.SemaphoreType.DMA,
            pltpu.SemaphoreType.DMA,
            pltpu.SemaphoreType.DMA,
            pltpu.SemaphoreType.DMA,
            pltpu.SemaphoreType.DMA,
            pltpu.SemaphoreType.DMA,
            pltpu.SemaphoreType.DMA,
            pltpu.SemaphoreType.DMA,
        ],
    )
    def spmm(rows_hbm, cols_hbm, vals_hbm, dense_hbm, out_hbm,
             col_v, ridx0, ridx1, vbuf0, vbuf1, buf0, buf1, acc,
             gsem0, gsem1, ssem0, ssem1, rsem0, rsem1, vsem0, vsem1):
        cid = lax.axis_index("c")
        sid = lax.axis_index("s")
        wid = cid * NS + sid
        bufs = (buf0, buf1)
        ridxs = (ridx0, ridx1)
        vbufs = (vbuf0, vbuf1)
        gsems = (gsem0, gsem1)
        ssems = (ssem0, ssem1)
        rsems = (rsem0, rsem1)
        vsems = (vsem0, vsem1)

        def zero_row(r, carry):
            for j in range(dv):
                buf0[r, pl.ds(j * L, L)] = jnp.zeros((L,), jnp.float32)
            return carry
        lax.fori_loop(0, K, zero_row, 0)
        for j in range(npt // K):
            pltpu.sync_copy(buf0, acc.at[pl.ds(sid * npt + j * K, K)])
        plsc.subcore_barrier()

        base = wid * epw
        pltpu.sync_copy(cols_hbm.at[pl.ds(base, epw)], col_v)

        def prefetch(c, b):
            off = base + c * K
            pltpu.async_copy(rows_hbm.at[pl.ds(off, K)], ridxs[b], rsems[b])
            pltpu.async_copy(vals_hbm.at[pl.ds(off, K)], vbufs[b], vsems[b])

        def gather(c, b):
            pltpu.async_copy(dense_hbm.at[col_v.at[pl.ds(c * K, K)]],
                             bufs[b], gsems[b])

        def scale(b):
            buf = bufs[b]
            vbuf = vbufs[b]

            def scale_grp(g, carry):
                vgrp = vbuf[pl.ds(g * L, L)]
                for i in range(L):
                    vv = vgrp[i]
                    r = g * L + i
                    for j in range(dv):
                        buf[r, pl.ds(j * L, L)] = buf[r, pl.ds(j * L, L)] * vv
                return carry
            lax.fori_loop(0, K // L, scale_grp, 0)

        def scatter(b):
            pltpu.async_copy(bufs[b], acc.at[ridxs[b]], ssems[b],
                             add=True)

        def wait_in(b):
            pltpu.make_async_copy(dense_hbm.at[pl.ds(0, K)],
                                  bufs[b], gsems[b]).wait()
            pltpu.make_async_copy(rows_hbm.at[pl.ds(0, K)],
                                  ridxs[b], rsems[b]).wait()
            pltpu.make_async_copy(vals_hbm.at[pl.ds(0, K)],
                                  vbufs[b], vsems[b]).wait()

        def wait_scatter(b):
            pltpu.make_async_copy(dense_hbm.at[pl.ds(0, K)],
                                  bufs[b], ssems[b]).wait()

        prefetch(0, 0)
        gather(0, 0)
        prefetch(1, 1)
        gather(1, 1)
        wait_in(0)
        scale(0)
        scatter(0)

        def pair_body(i, carry):
            for b in range(2):
                c = 1 + 2 * i + b
                cb = (1 + b) % 2
                wait_scatter(1 - cb)
                prefetch(c + 1, 1 - cb)
                gather(c + 1, 1 - cb)
                wait_in(cb)
                scale(cb)
                scatter(cb)
            return carry
        lax.fori_loop(0, (n_chunks - 2) // 2, pair_body, 0)

        cb = (n_chunks - 1) % 2
        wait_scatter(1 - cb)
        wait_in(cb)
        scale(cb)
        scatter(cb)
        wait_scatter(cb)

        plsc.subcore_barrier()
        pltpu.sync_copy(acc.at[pl.ds(sid * npt, npt)],
                        out_hbm.at[cid, pl.ds(sid * npt, npt)])

    return spmm(rows, cols, vals, dense)


def _combine(partials, bias, n_rows):
    d = partials.shape[-1]
    blk = 1000
    assert n_rows % blk == 0

    if bias is None:
        def body(p_ref, o_ref):
            o_ref[...] = p_ref[0] + p_ref[1]
        in_specs = [pl.BlockSpec((NC, blk, d), lambda i: (0, i, 0))]
        operands = (partials,)
    else:
        def body(p_ref, b_ref, o_ref):
            o_ref[...] = p_ref[0] + p_ref[1] + b_ref[...]
        in_specs = [pl.BlockSpec((NC, blk, d), lambda i: (0, i, 0)),
                    pl.BlockSpec((1, d), lambda i: (0, 0))]
        operands = (partials, bias.reshape(1, d))

    return pl.pallas_call(
        body,
        grid=(n_rows // blk,),
        in_specs=in_specs,
        out_specs=pl.BlockSpec((blk, d), lambda i: (i, 0)),
        out_shape=jax.ShapeDtypeStruct((n_rows, d), jnp.float32),
    )(*operands)


def _pad_edges(indices, values, e_pad):
    e = values.shape[0]
    rows, cols, vals = indices[0], indices[1], values
    if e != e_pad:
        pad = e_pad - e
        rows = jnp.concatenate([rows, jnp.zeros((pad,), jnp.int32)])
        cols = jnp.concatenate([cols, jnp.zeros((pad,), jnp.int32)])
        vals = jnp.concatenate([vals, jnp.zeros((pad,), jnp.float32)])
    return rows, cols, vals


def kernel(x, pt_indices, pt_values, pd_indices, pd_values, W, b):
    n, _ = x.shape
    d = W.shape[1]
    e = pt_values.shape[0]

    grain = NW * K * 2
    e_pad = -(-e // grain) * grain
    n_pad = -(-n // (NS * K)) * (NS * K)

    pt_rows, pt_cols, pt_vals = _pad_edges(pt_indices, pt_values, e_pad)
    pd_rows, pd_cols, pd_vals = _pad_edges(pd_indices, pd_values, e_pad)

    support = _matmul(x, W)
    p1 = _spmm_partials(pt_rows, pt_cols, pt_vals, support, n_pad, d)
    midpu = _combine(p1, None, n)
    p2 = _spmm_partials(pd_rows, pd_cols, pd_vals, midpu, n_pad, d)
    return _combine(p2, b, n)


# speedup vs baseline: 1.2357x; 1.2357x over previous
import functools

import jax
import jax.numpy as jnp
from jax import lax
from jax.experimental import pallas as pl
from jax.experimental.pallas import tpu as pltpu
from jax.experimental.pallas import tpu_sc as plsc

NC = 2
NS = 16
NW = NC * NS
K = 64
L = 16


def _matmul(x, W):
    n, d_in = x.shape
    d_out = W.shape[1]
    blk = 1000

    def mm(x_ref, w_ref, o_ref):
        o_ref[...] = jnp.dot(x_ref[...], w_ref[...],
                             preferred_element_type=jnp.float32)

    return pl.pallas_call(
        mm,
        grid=(n // blk,),
        in_specs=[pl.BlockSpec((blk, d_in), lambda i: (i, 0)),
                  pl.BlockSpec((d_in, d_out), lambda i: (0, 0))],
        out_specs=pl.BlockSpec((blk, d_out), lambda i: (i, 0)),
        out_shape=jax.ShapeDtypeStruct((n, d_out), jnp.float32),
    )(x, W)


def _spmm_partials(rows, cols, vals, dense, n_pad, d, nc0, nc1):
    assert nc0 % 2 == 0 and nc1 % 2 == 0 and nc0 >= 4 and nc1 >= 4
    ncm = max(nc0, nc1)
    npt = n_pad // NS
    dv = d // L

    mesh = plsc.VectorSubcoreMesh(core_axis_name="c", subcore_axis_name="s")

    @functools.partial(
        pl.kernel,
        mesh=mesh,
        out_type=jax.ShapeDtypeStruct((NC, n_pad, d), jnp.float32),
        scratch_types=[
            pltpu.VMEM((ncm * K,), jnp.int32),
            pltpu.VMEM((K,), jnp.int32),
            pltpu.VMEM((K,), jnp.int32),
            pltpu.VMEM((K,), jnp.float32),
            pltpu.VMEM((K,), jnp.float32),
            pltpu.VMEM((K, d), jnp.float32),
            pltpu.VMEM((K, d), jnp.float32),
            pltpu.VMEM_SHARED((n_pad, d), jnp.float32),
            pltpu.SemaphoreType.DMA,
            pltpu.SemaphoreType.DMA,
            pltpu.SemaphoreType.DMA,
            pltpu.SemaphoreType.DMA,
            pltpu.SemaphoreType.DMA,
            pltpu.SemaphoreType.DMA,
            pltpu.SemaphoreType.DMA,
            pltpu.SemaphoreType.DMA,
        ],
    )
    def spmm(rows_hbm, cols_hbm, vals_hbm, dense_hbm, out_hbm,
             col_v, ridx0, ridx1, vbuf0, vbuf1, buf0, buf1, acc,
             gsem0, gsem1, ssem0, ssem1, rsem0, rsem1, vsem0, vsem1):
        cid = lax.axis_index("c")
        sid = lax.axis_index("s")
        nc = jnp.where(cid == 0, nc0, nc1)
        bufs = (buf0, buf1)
        ridxs = (ridx0, ridx1)
        vbufs = (vbuf0, vbuf1)
        gsems = (gsem0, gsem1)
        ssems = (ssem0, ssem1)
        rsems = (rsem0, rsem1)
        vsems = (vsem0, vsem1)

        def zero_row(r, carry):
            for j in range(dv):
                buf0[r, pl.ds(j * L, L)] = jnp.zeros((L,), jnp.float32)
            return carry
        lax.fori_loop(0, K, zero_row, 0)
        for j in range(npt // K):
            pltpu.sync_copy(buf0, acc.at[pl.ds(sid * npt + j * K, K)])
        plsc.subcore_barrier()

        base = cid * (NS * nc0 * K) + sid * (nc * K)
        pltpu.sync_copy(cols_hbm.at[pl.ds(base, ncm * K)], col_v)

        def prefetch(c, b):
            off = base + c * K
            pltpu.async_copy(rows_hbm.at[pl.ds(off, K)], ridxs[b], rsems[b])
            pltpu.async_copy(vals_hbm.at[pl.ds(off, K)], vbufs[b], vsems[b])

        def gather(c, b):
            pltpu.async_copy(dense_hbm.at[col_v.at[pl.ds(c * K, K)]],
                             bufs[b], gsems[b])

        def scale(b):
            buf = bufs[b]
            vbuf = vbufs[b]

            def scale_grp(g, carry):
                vgrp = vbuf[pl.ds(g * L, L)]
                for i in range(L):
                    vv = vgrp[i]
                    r = g * L + i
                    for j in range(dv):
                        buf[r, pl.ds(j * L, L)] = buf[r, pl.ds(j * L, L)] * vv
                return carry
            lax.fori_loop(0, K // L, scale_grp, 0)

        def scatter(b):
            pltpu.async_copy(bufs[b], acc.at[ridxs[b]], ssems[b],
                             add=True)

        def wait_in(b):
            pltpu.make_async_copy(dense_hbm.at[pl.ds(0, K)],
                                  bufs[b], gsems[b]).wait()
            pltpu.make_async_copy(rows_hbm.at[pl.ds(0, K)],
                                  ridxs[b], rsems[b]).wait()
            pltpu.make_async_copy(vals_hbm.at[pl.ds(0, K)],
                                  vbufs[b], vsems[b]).wait()

        def wait_scatter(b):
            pltpu.make_async_copy(dense_hbm.at[pl.ds(0, K)],
                                  bufs[b], ssems[b]).wait()

        prefetch(0, 0)
        gather(0, 0)
        prefetch(1, 1)
        gather(1, 1)
        wait_in(0)
        scale(0)
        scatter(0)

        def pair_body(i, carry):
            for b in range(2):
                c = 1 + 2 * i + b
                cb = (1 + b) % 2
                wait_scatter(1 - cb)
                prefetch(c + 1, 1 - cb)
                gather(c + 1, 1 - cb)
                wait_in(cb)
                scale(cb)
                scatter(cb)
            return carry
        lax.fori_loop(0, (nc - 2) // 2, pair_body, 0)

        wait_scatter(0)
        wait_in(1)
        scale(1)
        scatter(1)
        wait_scatter(1)

        plsc.subcore_barrier()
        pltpu.sync_copy(acc.at[pl.ds(sid * npt, npt)],
                        out_hbm.at[cid, pl.ds(sid * npt, npt)])

    return spmm(rows, cols, vals, dense)


def _combine(partials, bias, n_rows):
    d = partials.shape[-1]
    blk = 1000
    assert n_rows % blk == 0

    if bias is None:
        def body(p_ref, o_ref):
            o_ref[...] = p_ref[0] + p_ref[1]
        in_specs = [pl.BlockSpec((NC, blk, d), lambda i: (0, i, 0))]
        operands = (partials,)
    else:
        def body(p_ref, b_ref, o_ref):
            o_ref[...] = p_ref[0] + p_ref[1] + b_ref[...]
        in_specs = [pl.BlockSpec((NC, blk, d), lambda i: (0, i, 0)),
                    pl.BlockSpec((1, d), lambda i: (0, 0))]
        operands = (partials, bias.reshape(1, d))

    return pl.pallas_call(
        body,
        grid=(n_rows // blk,),
        in_specs=in_specs,
        out_specs=pl.BlockSpec((blk, d), lambda i: (i, 0)),
        out_shape=jax.ShapeDtypeStruct((n_rows, d), jnp.float32),
    )(*operands)


def _pad_edges(indices, values, e_pad):
    e = values.shape[0]
    rows, cols, vals = indices[0], indices[1], values
    if e != e_pad:
        pad = e_pad - e
        rows = jnp.concatenate([rows, jnp.zeros((pad,), jnp.int32)])
        cols = jnp.concatenate([cols, jnp.zeros((pad,), jnp.int32)])
        vals = jnp.concatenate([vals, jnp.zeros((pad,), jnp.float32)])
    return rows, cols, vals


SC0_FRACTION = 0.355


def kernel(x, pt_indices, pt_values, pd_indices, pd_values, W, b):
    n, _ = x.shape
    d = W.shape[1]
    e = pt_values.shape[0]

    t_chunks = 2 * (-(-e // (NS * K * 2)))
    nc0 = max(4, 2 * int(round(t_chunks * SC0_FRACTION / 2)))
    nc1 = t_chunks - nc0
    assert nc1 >= 4
    e_pad = NS * t_chunks * K + (max(nc0, nc1) - min(nc0, nc1)) * K
    n_pad = -(-n // (NS * K)) * (NS * K)

    pt_rows, pt_cols, pt_vals = _pad_edges(pt_indices, pt_values, e_pad)
    pd_rows, pd_cols, pd_vals = _pad_edges(pd_indices, pd_values, e_pad)

    support = _matmul(x, W)
    p1 = _spmm_partials(pt_rows, pt_cols, pt_vals, support, n_pad, d,
                        nc0, nc1)
    midpu = _combine(p1, None, n)
    p2 = _spmm_partials(pd_rows, pd_cols, pd_vals, midpu, n_pad, d,
                        nc0, nc1)
    return _combine(p2, b, n)


# speedup vs baseline: 1.4829x; 1.2001x over previous
import functools

import jax
import jax.numpy as jnp
from jax import lax
from jax.experimental import pallas as pl
from jax.experimental.pallas import tpu as pltpu
from jax.experimental.pallas import tpu_sc as plsc

NC = 2
NS = 16
NW = NC * NS
K = 64
L = 16


def _matmul(x, W):
    n, d_in = x.shape
    d_out = W.shape[1]
    blk = 1000

    def mm(x_ref, w_ref, o_ref):
        o_ref[...] = jnp.dot(x_ref[...], w_ref[...],
                             preferred_element_type=jnp.float32)

    return pl.pallas_call(
        mm,
        grid=(n // blk,),
        in_specs=[pl.BlockSpec((blk, d_in), lambda i: (i, 0)),
                  pl.BlockSpec((d_in, d_out), lambda i: (0, 0))],
        out_specs=pl.BlockSpec((blk, d_out), lambda i: (i, 0)),
        out_shape=jax.ShapeDtypeStruct((n, d_out), jnp.float32),
    )(x, W)


def _spmm_partials(rows, cols, vals, dense, n_pad, d, nc0, nc1):
    assert nc0 % 2 == 0 and nc1 % 2 == 0 and nc0 >= 4 and nc1 >= 4
    ncm = max(nc0, nc1)
    npt = n_pad // NS
    dv = d // L

    mesh = plsc.VectorSubcoreMesh(core_axis_name="c", subcore_axis_name="s")

    @functools.partial(
        pl.kernel,
        mesh=mesh,
        out_type=jax.ShapeDtypeStruct((NC, n_pad, d), jnp.float32),
        scratch_types=[
            pltpu.VMEM((ncm * K,), jnp.int32),
            pltpu.VMEM((K,), jnp.int32),
            pltpu.VMEM((K,), jnp.int32),
            pltpu.VMEM((K,), jnp.float32),
            pltpu.VMEM((K,), jnp.float32),
            pltpu.VMEM((K, d), jnp.float32),
            pltpu.VMEM((K, d), jnp.float32),
            pltpu.VMEM_SHARED((n_pad, d), jnp.float32),
            pltpu.SemaphoreType.DMA,
            pltpu.SemaphoreType.DMA,
            pltpu.SemaphoreType.DMA,
            pltpu.SemaphoreType.DMA,
            pltpu.SemaphoreType.DMA,
            pltpu.SemaphoreType.DMA,
            pltpu.SemaphoreType.DMA,
            pltpu.SemaphoreType.DMA,
        ],
    )
    def spmm(rows_hbm, cols_hbm, vals_hbm, dense_hbm, out_hbm,
             col_v, ridx0, ridx1, vbuf0, vbuf1, buf0, buf1, acc,
             gsem0, gsem1, ssem0, ssem1, rsem0, rsem1, vsem0, vsem1):
        cid = lax.axis_index("c")
        sid = lax.axis_index("s")
        nc = jnp.where(cid == 0, nc0, nc1)
        bufs = (buf0, buf1)
        ridxs = (ridx0, ridx1)
        vbufs = (vbuf0, vbuf1)
        gsems = (gsem0, gsem1)
        ssems = (ssem0, ssem1)
        rsems = (rsem0, rsem1)
        vsems = (vsem0, vsem1)

        def zero_row(r, carry):
            for j in range(dv):
                buf0[r, pl.ds(j * L, L)] = jnp.zeros((L,), jnp.float32)
            return carry
        lax.fori_loop(0, K, zero_row, 0)
        for j in range(npt // K):
            pltpu.sync_copy(buf0, acc.at[pl.ds(sid * npt + j * K, K)])
        plsc.subcore_barrier()

        base = cid * (NS * nc0 * K) + sid * (nc * K)
        pltpu.sync_copy(cols_hbm.at[pl.ds(base, ncm * K)], col_v)

        def prefetch(c, b):
            off = base + c * K
            pltpu.async_copy(rows_hbm.at[pl.ds(off, K)], ridxs[b], rsems[b])
            pltpu.async_copy(vals_hbm.at[pl.ds(off, K)], vbufs[b], vsems[b])

        def gather(c, b):
            pltpu.async_copy(dense_hbm.at[col_v.at[pl.ds(c * K, K)]],
                             bufs[b], gsems[b])

        def scale(b):
            buf = bufs[b]
            vbuf = vbufs[b]

            def scale_grp(g, carry):
                vgrp = vbuf[pl.ds(g * L, L)]
                for i in range(L):
                    vv = vgrp[i]
                    r = g * L + i
                    for j in range(dv):
                        buf[r, pl.ds(j * L, L)] = buf[r, pl.ds(j * L, L)] * vv
                return carry
            lax.fori_loop(0, K // L, scale_grp, 0)

        def scatter(b):
            pltpu.async_copy(bufs[b], acc.at[ridxs[b]], ssems[b],
                             add=True)

        def wait_in(b):
            pltpu.make_async_copy(dense_hbm.at[pl.ds(0, K)],
                                  bufs[b], gsems[b]).wait()
            pltpu.make_async_copy(rows_hbm.at[pl.ds(0, K)],
                                  ridxs[b], rsems[b]).wait()
            pltpu.make_async_copy(vals_hbm.at[pl.ds(0, K)],
                                  vbufs[b], vsems[b]).wait()

        def wait_scatter(b):
            pltpu.make_async_copy(dense_hbm.at[pl.ds(0, K)],
                                  bufs[b], ssems[b]).wait()

        prefetch(0, 0)
        gather(0, 0)
        prefetch(1, 1)
        gather(1, 1)
        wait_in(0)
        scale(0)
        scatter(0)

        def pair_body(i, carry):
            for b in range(2):
                c = 1 + 2 * i + b
                cb = (1 + b) % 2
                wait_scatter(1 - cb)
                prefetch(c + 1, 1 - cb)
                gather(c + 1, 1 - cb)
                wait_in(cb)
                scale(cb)
                scatter(cb)
            return carry
        lax.fori_loop(0, (nc - 2) // 2, pair_body, 0)

        wait_scatter(0)
        wait_in(1)
        scale(1)
        scatter(1)
        wait_scatter(1)

        plsc.subcore_barrier()
        pltpu.sync_copy(acc.at[pl.ds(sid * npt, npt)],
                        out_hbm.at[cid, pl.ds(sid * npt, npt)])

    return spmm(rows, cols, vals, dense)


def _combine(partials, bias, n_rows):
    d = partials.shape[-1]
    blk = 1000
    assert n_rows % blk == 0

    if bias is None:
        def body(p_ref, o_ref):
            o_ref[...] = p_ref[0] + p_ref[1]
        in_specs = [pl.BlockSpec((NC, blk, d), lambda i: (0, i, 0))]
        operands = (partials,)
    else:
        def body(p_ref, b_ref, o_ref):
            o_ref[...] = p_ref[0] + p_ref[1] + b_ref[...]
        in_specs = [pl.BlockSpec((NC, blk, d), lambda i: (0, i, 0)),
                    pl.BlockSpec((1, d), lambda i: (0, 0))]
        operands = (partials, bias.reshape(1, d))

    return pl.pallas_call(
        body,
        grid=(n_rows // blk,),
        in_specs=in_specs,
        out_specs=pl.BlockSpec((blk, d), lambda i: (i, 0)),
        out_shape=jax.ShapeDtypeStruct((n_rows, d), jnp.float32),
    )(*operands)


def _pad_edges(indices, values, e_pad, n):
    e = values.shape[0]
    rows, cols, vals = indices[0], indices[1], values
    if e != e_pad:
        pad = e_pad - e
        spread = (jnp.arange(pad, dtype=jnp.int32) % n).astype(jnp.int32)
        rows = jnp.concatenate([rows, spread])
        cols = jnp.concatenate([cols, spread])
        vals = jnp.concatenate([vals, jnp.zeros((pad,), jnp.float32)])
    return rows, cols, vals


SC0_FRACTION = 0.355


def kernel(x, pt_indices, pt_values, pd_indices, pd_values, W, b):
    n, _ = x.shape
    d = W.shape[1]
    e = pt_values.shape[0]

    t_chunks = 2 * (-(-e // (NS * K * 2)))
    nc0 = max(4, 2 * int(round(t_chunks * SC0_FRACTION / 2)))
    nc1 = t_chunks - nc0
    assert nc1 >= 4
    e_pad = NS * t_chunks * K + (max(nc0, nc1) - min(nc0, nc1)) * K
    n_pad = -(-n // (NS * K)) * (NS * K)

    pt_rows, pt_cols, pt_vals = _pad_edges(pt_indices, pt_values, e_pad, n)
    pd_rows, pd_cols, pd_vals = _pad_edges(pd_indices, pd_values, e_pad, n)

    support = _matmul(x, W)
    p1 = _spmm_partials(pt_rows, pt_cols, pt_vals, support, n_pad, d,
                        nc0, nc1)
    midpu = _combine(p1, None, n)
    p2 = _spmm_partials(pd_rows, pd_cols, pd_vals, midpu, n_pad, d,
                        nc0, nc1)
    return _combine(p2, b, n)


# speedup vs baseline: 1.7794x; 1.2000x over previous
import functools

import jax
import jax.numpy as jnp
from jax import lax
from jax.experimental import pallas as pl
from jax.experimental.pallas import tpu as pltpu
from jax.experimental.pallas import tpu_sc as plsc

NC = 2
NS = 16
NW = NC * NS
K = 64
L = 16


def _matmul(x, W):
    n, d_in = x.shape
    d_out = W.shape[1]
    blk = 1000

    def mm(x_ref, w_ref, o_ref):
        o_ref[...] = jnp.dot(x_ref[...], w_ref[...],
                             preferred_element_type=jnp.float32)

    return pl.pallas_call(
        mm,
        grid=(n // blk,),
        in_specs=[pl.BlockSpec((blk, d_in), lambda i: (i, 0)),
                  pl.BlockSpec((d_in, d_out), lambda i: (0, 0))],
        out_specs=pl.BlockSpec((blk, d_out), lambda i: (i, 0)),
        out_shape=jax.ShapeDtypeStruct((n, d_out), jnp.float32),
    )(x, W)


def _spmm_partials(rows, cols, vals, dense, n_pad, d, nc0, nc1):
    assert nc0 % 2 == 0 and nc1 % 2 == 0 and nc0 >= 4 and nc1 >= 4
    ncm = max(nc0, nc1)
    npt = n_pad // NS
    dv = d // L

    mesh = plsc.VectorSubcoreMesh(core_axis_name="c", subcore_axis_name="s")

    @functools.partial(
        pl.kernel,
        mesh=mesh,
        out_type=jax.ShapeDtypeStruct((NC, n_pad, d), jnp.float32),
        scratch_types=[
            pltpu.VMEM((ncm * K,), jnp.int32),
            pltpu.VMEM((K,), jnp.int32),
            pltpu.VMEM((K,), jnp.int32),
            pltpu.VMEM((K,), jnp.float32),
            pltpu.VMEM((K,), jnp.float32),
            pltpu.VMEM((K, d), jnp.float32),
            pltpu.VMEM((K, d), jnp.float32),
            pltpu.VMEM_SHARED((n_pad, d), jnp.float32),
            pltpu.SemaphoreType.DMA,
            pltpu.SemaphoreType.DMA,
            pltpu.SemaphoreType.DMA,
            pltpu.SemaphoreType.DMA,
            pltpu.SemaphoreType.DMA,
            pltpu.SemaphoreType.DMA,
            pltpu.SemaphoreType.DMA,
            pltpu.SemaphoreType.DMA,
        ],
    )
    def spmm(rows_hbm, cols_hbm, vals_hbm, dense_hbm, out_hbm,
             col_v, ridx0, ridx1, vbuf0, vbuf1, buf0, buf1, acc,
             gsem0, gsem1, ssem0, ssem1, rsem0, rsem1, vsem0, vsem1):
        cid = lax.axis_index("c")
        sid = lax.axis_index("s")
        nc = jnp.where(cid == 0, nc0, nc1)
        bufs = (buf0, buf1)
        ridxs = (ridx0, ridx1)
        vbufs = (vbuf0, vbuf1)
        gsems = (gsem0, gsem1)
        ssems = (ssem0, ssem1)
        rsems = (rsem0, rsem1)
        vsems = (vsem0, vsem1)

        def zero_row(r, carry):
            for j in range(dv):
                buf0[r, pl.ds(j * L, L)] = jnp.zeros((L,), jnp.float32)
            return carry
        lax.fori_loop(0, K, zero_row, 0)
        for j in range(npt // K):
            pltpu.sync_copy(buf0, acc.at[pl.ds(sid * npt + j * K, K)])
        plsc.subcore_barrier()

        base = cid * (NS * nc0 * K) + sid * (nc * K)
        pltpu.sync_copy(cols_hbm.at[pl.ds(base, ncm * K)], col_v)

        def prefetch(c, b):
            off = base + c * K
            pltpu.async_copy(rows_hbm.at[pl.ds(off, K)], ridxs[b], rsems[b])
            pltpu.async_copy(vals_hbm.at[pl.ds(off, K)], vbufs[b], vsems[b])

        def gather(c, b):
            pltpu.async_copy(dense_hbm.at[col_v.at[pl.ds(c * K, K)]],
                             bufs[b], gsems[b])

        def scale(b):
            buf = bufs[b]
            vbuf = vbufs[b]

            def scale_grp(g, carry):
                vgrp = vbuf[pl.ds(g * L, L)]
                for i in range(L):
                    vv = vgrp[i]
                    r = g * L + i
                    for j in range(dv):
                        buf[r, pl.ds(j * L, L)] = buf[r, pl.ds(j * L, L)] * vv
                return carry
            lax.fori_loop(0, K // L, scale_grp, 0)

        def scatter(b):
            pltpu.async_copy(bufs[b], acc.at[ridxs[b]], ssems[b],
                             add=True)

        def wait_in(b):
            pltpu.make_async_copy(dense_hbm.at[pl.ds(0, K)],
                                  bufs[b], gsems[b]).wait()
            pltpu.make_async_copy(rows_hbm.at[pl.ds(0, K)],
                                  ridxs[b], rsems[b]).wait()
            pltpu.make_async_copy(vals_hbm.at[pl.ds(0, K)],
                                  vbufs[b], vsems[b]).wait()

        def wait_scatter(b):
            pltpu.make_async_copy(dense_hbm.at[pl.ds(0, K)],
                                  bufs[b], ssems[b]).wait()

        prefetch(0, 0)
        gather(0, 0)
        prefetch(1, 1)
        gather(1, 1)
        wait_in(0)
        scale(0)
        scatter(0)

        def pair_body(i, carry):
            for b in range(2):
                c = 1 + 2 * i + b
                cb = (1 + b) % 2
                wait_scatter(1 - cb)
                prefetch(c + 1, 1 - cb)
                gather(c + 1, 1 - cb)
                wait_in(cb)
                scale(cb)
                scatter(cb)
            return carry
        lax.fori_loop(0, (nc - 2) // 2, pair_body, 0)

        wait_scatter(0)
        wait_in(1)
        scale(1)
        scatter(1)
        wait_scatter(1)

        plsc.subcore_barrier()
        pltpu.sync_copy(acc.at[pl.ds(sid * npt, npt)],
                        out_hbm.at[cid, pl.ds(sid * npt, npt)])

    return spmm(rows, cols, vals, dense)


def _combine(partials, bias, n_rows):
    d = partials.shape[-1]
    blk = 1000
    assert n_rows % blk == 0

    if bias is None:
        def body(p_ref, o_ref):
            o_ref[...] = p_ref[0] + p_ref[1]
        in_specs = [pl.BlockSpec((NC, blk, d), lambda i: (0, i, 0))]
        operands = (partials,)
    else:
        def body(p_ref, b_ref, o_ref):
            o_ref[...] = p_ref[0] + p_ref[1] + b_ref[...]
        in_specs = [pl.BlockSpec((NC, blk, d), lambda i: (0, i, 0)),
                    pl.BlockSpec((1, d), lambda i: (0, 0))]
        operands = (partials, bias.reshape(1, d))

    return pl.pallas_call(
        body,
        grid=(n_rows // blk,),
        in_specs=in_specs,
        out_specs=pl.BlockSpec((blk, d), lambda i: (i, 0)),
        out_shape=jax.ShapeDtypeStruct((n_rows, d), jnp.float32),
    )(*operands)


def _pad_edges(indices, values, e_pad, n):
    e = values.shape[0]
    rows, cols, vals = indices[0], indices[1], values
    if e != e_pad:
        pad = e_pad - e
        spread = (jnp.arange(pad, dtype=jnp.int32) % n).astype(jnp.int32)
        rows = jnp.concatenate([rows, spread])
        cols = jnp.concatenate([cols, spread])
        vals = jnp.concatenate([vals, jnp.zeros((pad,), jnp.float32)])
    return rows, cols, vals


SC0_FRACTION = 0.5


def kernel(x, pt_indices, pt_values, pd_indices, pd_values, W, b):
    n, _ = x.shape
    d = W.shape[1]
    e = pt_values.shape[0]

    t_chunks = 2 * (-(-e // (NS * K * 2)))
    nc0 = max(4, 2 * int(round(t_chunks * SC0_FRACTION / 2)))
    nc1 = t_chunks - nc0
    assert nc1 >= 4
    e_pad = NS * t_chunks * K + (max(nc0, nc1) - min(nc0, nc1)) * K
    n_pad = -(-n // (NS * K)) * (NS * K)

    pt_rows, pt_cols, pt_vals = _pad_edges(pt_indices, pt_values, e_pad, n)
    pd_rows, pd_cols, pd_vals = _pad_edges(pd_indices, pd_values, e_pad, n)

    support = _matmul(x, W)
    p1 = _spmm_partials(pt_rows, pt_cols, pt_vals, support, n_pad, d,
                        nc0, nc1)
    midpu = _combine(p1, None, n)
    p2 = _spmm_partials(pd_rows, pd_cols, pd_vals, midpu, n_pad, d,
                        nc0, nc1)
    return _combine(p2, b, n)


# speedup vs baseline: 1.9757x; 1.1103x over previous
import functools

import jax
import jax.numpy as jnp
from jax import lax
from jax.experimental import pallas as pl
from jax.experimental.pallas import tpu as pltpu
from jax.experimental.pallas import tpu_sc as plsc

NC = 2
NS = 16
NW = NC * NS
K = 128
L = 16


def _matmul(x, W):
    n, d_in = x.shape
    d_out = W.shape[1]
    blk = 1000

    def mm(x_ref, w_ref, o_ref):
        o_ref[...] = jnp.dot(x_ref[...], w_ref[...],
                             preferred_element_type=jnp.float32)

    return pl.pallas_call(
        mm,
        grid=(n // blk,),
        in_specs=[pl.BlockSpec((blk, d_in), lambda i: (i, 0)),
                  pl.BlockSpec((d_in, d_out), lambda i: (0, 0))],
        out_specs=pl.BlockSpec((blk, d_out), lambda i: (i, 0)),
        out_shape=jax.ShapeDtypeStruct((n, d_out), jnp.float32),
    )(x, W)


def _spmm_partials(rows, cols, vals, dense, n_pad, d, nc0, nc1):
    assert nc0 % 2 == 0 and nc1 % 2 == 0 and nc0 >= 4 and nc1 >= 4
    ncm = max(nc0, nc1)
    npt = n_pad // NS
    dv = d // L

    mesh = plsc.VectorSubcoreMesh(core_axis_name="c", subcore_axis_name="s")

    @functools.partial(
        pl.kernel,
        mesh=mesh,
        out_type=jax.ShapeDtypeStruct((NC, n_pad, d), jnp.float32),
        scratch_types=[
            pltpu.VMEM((ncm * K,), jnp.int32),
            pltpu.VMEM((K,), jnp.int32),
            pltpu.VMEM((K,), jnp.int32),
            pltpu.VMEM((K,), jnp.float32),
            pltpu.VMEM((K,), jnp.float32),
            pltpu.VMEM((K, d), jnp.float32),
            pltpu.VMEM((K, d), jnp.float32),
            pltpu.VMEM_SHARED((n_pad, d), jnp.float32),
            pltpu.SemaphoreType.DMA,
            pltpu.SemaphoreType.DMA,
            pltpu.SemaphoreType.DMA,
            pltpu.SemaphoreType.DMA,
            pltpu.SemaphoreType.DMA,
            pltpu.SemaphoreType.DMA,
            pltpu.SemaphoreType.DMA,
            pltpu.SemaphoreType.DMA,
        ],
    )
    def spmm(rows_hbm, cols_hbm, vals_hbm, dense_hbm, out_hbm,
             col_v, ridx0, ridx1, vbuf0, vbuf1, buf0, buf1, acc,
             gsem0, gsem1, ssem0, ssem1, rsem0, rsem1, vsem0, vsem1):
        cid = lax.axis_index("c")
        sid = lax.axis_index("s")
        nc = jnp.where(cid == 0, nc0, nc1)
        bufs = (buf0, buf1)
        ridxs = (ridx0, ridx1)
        vbufs = (vbuf0, vbuf1)
        gsems = (gsem0, gsem1)
        ssems = (ssem0, ssem1)
        rsems = (rsem0, rsem1)
        vsems = (vsem0, vsem1)

        def zero_row(r, carry):
            for j in range(dv):
                buf0[r, pl.ds(j * L, L)] = jnp.zeros((L,), jnp.float32)
            return carry
        lax.fori_loop(0, K, zero_row, 0)
        for j in range(npt // K):
            pltpu.sync_copy(buf0, acc.at[pl.ds(sid * npt + j * K, K)])
        if npt % K:
            pltpu.sync_copy(buf0.at[pl.ds(0, npt % K)],
                            acc.at[pl.ds(sid * npt + (npt // K) * K, npt % K)])
        plsc.subcore_barrier()

        base = cid * (NS * nc0 * K) + sid * (nc * K)
        pltpu.sync_copy(cols_hbm.at[pl.ds(base, ncm * K)], col_v)

        def prefetch(c, b):
            off = base + c * K
            pltpu.async_copy(rows_hbm.at[pl.ds(off, K)], ridxs[b], rsems[b])
            pltpu.async_copy(vals_hbm.at[pl.ds(off, K)], vbufs[b], vsems[b])

        def gather(c, b):
            pltpu.async_copy(dense_hbm.at[col_v.at[pl.ds(c * K, K)]],
                             bufs[b], gsems[b])

        def scale(b):
            buf = bufs[b]
            vbuf = vbufs[b]

            def scale_grp(g, carry):
                vgrp = vbuf[pl.ds(g * L, L)]
                for i in range(L):
                    vv = vgrp[i]
                    r = g * L + i
                    for j in range(dv):
                        buf[r, pl.ds(j * L, L)] = buf[r, pl.ds(j * L, L)] * vv
                return carry
            lax.fori_loop(0, K // L, scale_grp, 0)

        def scatter(b):
            pltpu.async_copy(bufs[b], acc.at[ridxs[b]], ssems[b],
                             add=True)

        def wait_in(b):
            pltpu.make_async_copy(dense_hbm.at[pl.ds(0, K)],
                                  bufs[b], gsems[b]).wait()
            pltpu.make_async_copy(rows_hbm.at[pl.ds(0, K)],
                                  ridxs[b], rsems[b]).wait()
            pltpu.make_async_copy(vals_hbm.at[pl.ds(0, K)],
                                  vbufs[b], vsems[b]).wait()

        def wait_scatter(b):
            pltpu.make_async_copy(dense_hbm.at[pl.ds(0, K)],
                                  bufs[b], ssems[b]).wait()

        prefetch(0, 0)
        gather(0, 0)
        prefetch(1, 1)
        gather(1, 1)
        wait_in(0)
        scale(0)
        scatter(0)

        def pair_body(i, carry):
            for b in range(2):
                c = 1 + 2 * i + b
                cb = (1 + b) % 2
                wait_scatter(1 - cb)
                prefetch(c + 1, 1 - cb)
                gather(c + 1, 1 - cb)
                wait_in(cb)
                scale(cb)
                scatter(cb)
            return carry
        lax.fori_loop(0, (nc - 2) // 2, pair_body, 0)

        wait_scatter(0)
        wait_in(1)
        scale(1)
        scatter(1)
        wait_scatter(1)

        plsc.subcore_barrier()
        pltpu.sync_copy(acc.at[pl.ds(sid * npt, npt)],
                        out_hbm.at[cid, pl.ds(sid * npt, npt)])

    return spmm(rows, cols, vals, dense)


def _combine(partials, bias, n_rows):
    d = partials.shape[-1]
    blk = 1000
    assert n_rows % blk == 0

    if bias is None:
        def body(p_ref, o_ref):
            o_ref[...] = p_ref[0] + p_ref[1]
        in_specs = [pl.BlockSpec((NC, blk, d), lambda i: (0, i, 0))]
        operands = (partials,)
    else:
        def body(p_ref, b_ref, o_ref):
            o_ref[...] = p_ref[0] + p_ref[1] + b_ref[...]
        in_specs = [pl.BlockSpec((NC, blk, d), lambda i: (0, i, 0)),
                    pl.BlockSpec((1, d), lambda i: (0, 0))]
        operands = (partials, bias.reshape(1, d))

    return pl.pallas_call(
        body,
        grid=(n_rows // blk,),
        in_specs=in_specs,
        out_specs=pl.BlockSpec((blk, d), lambda i: (i, 0)),
        out_shape=jax.ShapeDtypeStruct((n_rows, d), jnp.float32),
    )(*operands)


def _pad_edges(indices, values, e_pad, n):
    e = values.shape[0]
    rows, cols, vals = indices[0], indices[1], values
    if e != e_pad:
        pad = e_pad - e
        spread = (jnp.arange(pad, dtype=jnp.int32) % n).astype(jnp.int32)
        rows = jnp.concatenate([rows, spread])
        cols = jnp.concatenate([cols, spread])
        vals = jnp.concatenate([vals, jnp.zeros((pad,), jnp.float32)])
    return rows, cols, vals


SC0_FRACTION = 0.5


def kernel(x, pt_indices, pt_values, pd_indices, pd_values, W, b):
    n, _ = x.shape
    d = W.shape[1]
    e = pt_values.shape[0]

    t_chunks = 2 * (-(-e // (NS * K * 2)))
    nc0 = max(4, 2 * int(round(t_chunks * SC0_FRACTION / 2)))
    nc1 = t_chunks - nc0
    assert nc1 >= 4
    e_pad = NS * t_chunks * K + (max(nc0, nc1) - min(nc0, nc1)) * K
    n_pad = -(-n // (NS * 8)) * (NS * 8)

    pt_rows, pt_cols, pt_vals = _pad_edges(pt_indices, pt_values, e_pad, n)
    pd_rows, pd_cols, pd_vals = _pad_edges(pd_indices, pd_values, e_pad, n)

    support = _matmul(x, W)
    p1 = _spmm_partials(pt_rows, pt_cols, pt_vals, support, n_pad, d,
                        nc0, nc1)
    midpu = _combine(p1, None, n)
    p2 = _spmm_partials(pd_rows, pd_cols, pd_vals, midpu, n_pad, d,
                        nc0, nc1)
    return _combine(p2, b, n)
